# Initial kernel scaffold; baseline (speedup 1.0000x reference)
import jax
import jax.numpy as jnp
from jax.experimental import pallas as pl


def kernel(x, g0, g1, g2, edge_index, ws0, wn0, b0, ws1, wn1, b1, ws2, wn2, b2, ws3, wn3, b3, m1w1, m1b1, m1w2, m1b2, gamma, beta, m2w1, m2b1, m2w2, m2b2):
    raise NotImplementedError("write your pallas kernel here")


# speedup vs baseline: 5.8095x; 5.8095x over previous
import jax
import jax.numpy as jnp
from jax import lax
from jax.experimental import pallas as pl
from jax.experimental.pallas import tpu as pltpu
from jax.experimental.pallas import tpu_sc as plsc

N = 10000
E = 320000
D_X = 128
D_G = 32
IN_DIM = D_X + 3 * D_G
HID = 128
OUT_DIM = 8
MAX_NUM_NODES = 1000
BATCH = (N * HID) // (HID * MAX_NUM_NODES)

NC = 2
NS = 16
CW = 100
ROWS_TOT = E // CW
ROWS_PER_TILE = ROWS_TOT // NS
NPT = N // NS

_BN = 1000


def _seg_sum_body(ha_hbm, hb_hbm, srcc_hbm, dstc_hbm, zrows_hbm, out_hbm,
                  src_v, dst_v, rows_a, rows_b, agg_sh, sem_a, sem_b):
    c = lax.axis_index("c")
    s = lax.axis_index("s")
    pltpu.sync_copy(srcc_hbm.at[s], src_v)
    pltpu.sync_copy(dstc_hbm.at[s], dst_v)
    pltpu.sync_copy(zrows_hbm, agg_sh.at[pl.ds(s * NPT, NPT)])
    plsc.subcore_barrier()

    def edge_loop(h_hbm):
        pltpu.async_copy(h_hbm.at[src_v.at[0]], rows_a, sem_a)

        def chunk_pair(k, carry):
            j = 2 * k
            pltpu.make_async_copy(h_hbm.at[src_v.at[j]], rows_a, sem_a).wait()
            pltpu.async_copy(h_hbm.at[src_v.at[j + 1]], rows_b, sem_b)
            pltpu.sync_copy(rows_a, agg_sh.at[dst_v.at[j]], add=True)
            pltpu.make_async_copy(h_hbm.at[src_v.at[j + 1]], rows_b,
                                  sem_b).wait()

            @pl.when(k < ROWS_PER_TILE // 2 - 1)
            def _():
                pltpu.async_copy(h_hbm.at[src_v.at[j + 2]], rows_a, sem_a)

            pltpu.sync_copy(rows_b, agg_sh.at[dst_v.at[j + 1]], add=True)
            return carry

        lax.fori_loop(0, ROWS_PER_TILE // 2, chunk_pair, 0)

    @pl.when(c == 0)
    def _():
        edge_loop(ha_hbm)

    @pl.when(c == 1)
    def _():
        edge_loop(hb_hbm)

    plsc.subcore_barrier()
    pltpu.sync_copy(agg_sh.at[pl.ds(s * NPT, NPT)], out_hbm.at[c, s])


def _seg_sum(ha, hb, srcc, dstc, zrows):
    wh = ha.shape[1]
    mesh = plsc.VectorSubcoreMesh(core_axis_name="c", subcore_axis_name="s")
    f = pl.kernel(
        _seg_sum_body,
        out_type=jax.ShapeDtypeStruct((NC, NS, NPT, wh), jnp.float32),
        mesh=mesh,
        scratch_types=[
            pltpu.VMEM((ROWS_PER_TILE, CW), jnp.int32),
            pltpu.VMEM((ROWS_PER_TILE, CW), jnp.int32),
            pltpu.VMEM((CW, wh), jnp.float32),
            pltpu.VMEM((CW, wh), jnp.float32),
            pltpu.VMEM_SHARED((N, wh), jnp.float32),
            pltpu.SemaphoreType.DMA,
            pltpu.SemaphoreType.DMA,
        ],
        compiler_params=pltpu.CompilerParams(use_tc_tiling_on_sc=False),
    )
    out = f(ha, hb, srcc, dstc, zrows)
    return out[0].reshape(N, wh), out[1].reshape(N, wh)


def _deg_body(dstc_hbm, ones_hbm, zrows_hbm, out_hbm,
              dst_v, ones_v, deg_sh, sem):
    c = lax.axis_index("c")
    s = lax.axis_index("s")
    pltpu.sync_copy(dstc_hbm.at[s], dst_v)
    pltpu.sync_copy(ones_hbm, ones_v)
    pltpu.sync_copy(zrows_hbm, deg_sh.at[pl.ds(s * NPT, NPT)])
    plsc.subcore_barrier()

    half = ROWS_PER_TILE // NC

    def chunk(j, carry):
        pltpu.sync_copy(ones_v, deg_sh.at[dst_v.at[c * half + j]], add=True)
        return carry

    lax.fori_loop(0, half, chunk, 0)
    plsc.subcore_barrier()
    pltpu.sync_copy(deg_sh.at[pl.ds(s * NPT, NPT)], out_hbm.at[c, s])


def _deg_count(dstc, ones16, zrows16):
    mesh = plsc.VectorSubcoreMesh(core_axis_name="c", subcore_axis_name="s")
    f = pl.kernel(
        _deg_body,
        out_type=jax.ShapeDtypeStruct((NC, NS, NPT, 16), jnp.float32),
        mesh=mesh,
        scratch_types=[
            pltpu.VMEM((ROWS_PER_TILE, CW), jnp.int32),
            pltpu.VMEM((CW, 16), jnp.float32),
            pltpu.VMEM_SHARED((N, 16), jnp.float32),
            pltpu.SemaphoreType.DMA,
        ],
        compiler_params=pltpu.CompilerParams(use_tc_tiling_on_sc=False),
    )
    return f(dstc, ones16, zrows16).reshape(NC, N, 16)


def _layer_body(nh, na, *refs):
    h_parts = [refs[i][...] for i in range(nh)]
    agg_parts = [refs[nh + i][...] for i in range(na)]
    deg_ref, w_ref, b_ref = refs[nh + na:nh + na + 3]
    outs = refs[nh + na + 3:]
    h = jnp.concatenate(h_parts, axis=1)
    agg = jnp.concatenate(agg_parts, axis=1)
    deg = deg_ref[0, :, 0:1] + deg_ref[1, :, 0:1]
    mean = agg / jnp.maximum(deg, 1.0)
    w = w_ref[...]
    wh = h.shape[1]
    r = (jnp.dot(mean, w[wh:], preferred_element_type=jnp.float32)
         + jnp.dot(h, w[:wh], preferred_element_type=jnp.float32))
    r = jnp.maximum(r + b_ref[...], 0.0)
    if len(outs) == 1:
        outs[0][...] = r
    else:
        outs[0][...] = r[:, :HID // 2]
        outs[1][...] = r[:, HID // 2:]


def _layer(h_parts, agg_parts, deg2, wstack, bias, split_out):
    import functools
    w2 = wstack.shape[0]
    if split_out:
        out_specs = [pl.BlockSpec((_BN, HID // 2), lambda i: (i, 0)),
                     pl.BlockSpec((_BN, HID // 2), lambda i: (i, 0))]
        out_shape = [jax.ShapeDtypeStruct((N, HID // 2), jnp.float32),
                     jax.ShapeDtypeStruct((N, HID // 2), jnp.float32)]
    else:
        out_specs = pl.BlockSpec((_BN, HID), lambda i: (i, 0))
        out_shape = jax.ShapeDtypeStruct((N, HID), jnp.float32)
    in_specs = (
        [pl.BlockSpec((_BN, p.shape[1]), lambda i: (i, 0)) for p in h_parts]
        + [pl.BlockSpec((_BN, p.shape[1]), lambda i: (i, 0))
           for p in agg_parts]
        + [pl.BlockSpec((NC, _BN, 16), lambda i: (0, i, 0)),
           pl.BlockSpec((w2, HID), lambda i: (0, 0)),
           pl.BlockSpec((1, HID), lambda i: (0, 0))])
    body = functools.partial(_layer_body, len(h_parts), len(agg_parts))
    return pl.pallas_call(
        body,
        grid=(N // _BN,),
        in_specs=in_specs,
        out_specs=out_specs,
        out_shape=out_shape,
    )(*h_parts, *agg_parts, deg2, wstack, bias)


def _head1_body(hr_ref, w1_ref, b1_ref, w2_ref, b2_ref, u_out):
    t = jnp.dot(hr_ref[...], w1_ref[...],
                preferred_element_type=jnp.float32) + b1_ref[...]
    t = jnp.maximum(t, 0.0)
    u_out[...] = jnp.dot(t, w2_ref[...],
                         preferred_element_type=jnp.float32) + b2_ref[...]


def _head1(hr, m1w1, m1b1, m1w2, m1b2):
    R = BATCH * HID
    return pl.pallas_call(
        _head1_body,
        grid=(1,),
        in_specs=[
            pl.BlockSpec((R, MAX_NUM_NODES), lambda i: (0, 0)),
            pl.BlockSpec((MAX_NUM_NODES, HID), lambda i: (0, 0)),
            pl.BlockSpec((1, HID), lambda i: (0, 0)),
            pl.BlockSpec((HID, 1), lambda i: (0, 0)),
            pl.BlockSpec((1, 1), lambda i: (0, 0)),
        ],
        out_specs=pl.BlockSpec((R, 1), lambda i: (0, 0)),
        out_shape=jax.ShapeDtypeStruct((R, 1), jnp.float32),
    )(hr, m1w1, m1b1, m1w2, m1b2)


def _head2_body(u_ref, gamma_ref, beta_ref, w1_ref, b1_ref, w2_ref, b2_ref,
                out_ref):
    u = u_ref[...]
    mu = jnp.mean(u, axis=0, keepdims=True)
    var = jnp.mean((u - mu) ** 2, axis=0, keepdims=True)
    hb = (u - mu) / jnp.sqrt(var + 1e-5) * gamma_ref[...] + beta_ref[...]
    hb = jnp.maximum(hb, 0.0)
    z = jnp.dot(hb, w1_ref[...], preferred_element_type=jnp.float32)
    z = jnp.maximum(z + b1_ref[...], 0.0)
    out_ref[...] = jnp.dot(z, w2_ref[...],
                           preferred_element_type=jnp.float32) + b2_ref[...]


def _head2(u2, gamma, beta, m2w1, m2b1, m2w2, m2b2):
    return pl.pallas_call(
        _head2_body,
        grid=(1,),
        in_specs=[
            pl.BlockSpec((BATCH, HID), lambda i: (0, 0)),
            pl.BlockSpec((1, HID), lambda i: (0, 0)),
            pl.BlockSpec((1, HID), lambda i: (0, 0)),
            pl.BlockSpec((HID, HID), lambda i: (0, 0)),
            pl.BlockSpec((1, HID), lambda i: (0, 0)),
            pl.BlockSpec((HID, OUT_DIM), lambda i: (0, 0)),
            pl.BlockSpec((1, OUT_DIM), lambda i: (0, 0)),
        ],
        out_specs=pl.BlockSpec((BATCH, OUT_DIM), lambda i: (0, 0)),
        out_shape=jax.ShapeDtypeStruct((BATCH, OUT_DIM), jnp.float32),
    )(u2, gamma, beta, m2w1, m2b1, m2w2, m2b2)


def kernel(x, g0, g1, g2, edge_index, ws0, wn0, b0, ws1, wn1, b1,
           ws2, wn2, b2, ws3, wn3, b3, m1w1, m1b1, m1w2, m1b2,
           gamma, beta, m2w1, m2b1, m2w2, m2b2):
    srcc = edge_index[0].reshape(NS, ROWS_PER_TILE, CW)
    dstc = edge_index[1].reshape(NS, ROWS_PER_TILE, CW)
    zrows = jnp.zeros((NPT, HID // 2), jnp.float32)
    zrows16 = jnp.zeros((NPT, 16), jnp.float32)
    ones16 = jnp.ones((CW, 16), jnp.float32)

    deg2 = _deg_count(dstc, ones16, zrows16)

    h0 = jnp.concatenate([x, g0, g1, g2], axis=1)
    zcols = jnp.zeros((N, 2 * HID - IN_DIM), jnp.float32)
    h0p = jnp.concatenate([h0, zcols], axis=1)
    FH = HID // 2

    agg00, agg01 = _seg_sum(h0p[:, :FH], h0p[:, FH:2 * FH], srcc, dstc, zrows)
    agg02, agg03 = _seg_sum(h0p[:, 2 * FH:3 * FH], h0p[:, 3 * FH:],
                            srcc, dstc, zrows)
    wstack0 = jnp.concatenate(
        [ws0, wn0, jnp.zeros((2 * HID - IN_DIM, HID), jnp.float32)], axis=0)
    ha, hb = _layer([h0[:, :IN_DIM // 2], h0[:, IN_DIM // 2:]],
                    [agg00, agg01, agg02, agg03], deg2, wstack0,
                    b0.reshape(1, -1), split_out=True)

    h4 = None
    layers = ((ws1, wn1, b1), (ws2, wn2, b2), (ws3, wn3, b3))
    for li, (ws, wn, b) in enumerate(layers):
        agga, aggb = _seg_sum(ha, hb, srcc, dstc, zrows)
        wstack = jnp.concatenate([ws, wn], axis=0)
        last = li == len(layers) - 1
        r = _layer([ha, hb], [agga, aggb], deg2, wstack, b.reshape(1, -1),
                   split_out=not last)
        if last:
            h4 = r
        else:
            ha, hb = r

    hr = h4.reshape(BATCH * HID, MAX_NUM_NODES)
    u = _head1(hr, m1w1, m1b1.reshape(1, -1), m1w2, m1b2.reshape(1, 1))
    u2 = u.reshape(BATCH, HID)
    return _head2(u2, gamma.reshape(1, -1), beta.reshape(1, -1),
                  m2w1, m2b1.reshape(1, -1), m2w2, m2b2.reshape(1, -1))


# speedup vs baseline: 8.0026x; 1.3775x over previous
import jax
import jax.numpy as jnp
from jax import lax
from jax.experimental import pallas as pl
from jax.experimental.pallas import tpu as pltpu
from jax.experimental.pallas import tpu_sc as plsc

N = 10000
E = 320000
D_X = 128
D_G = 32
IN_DIM = D_X + 3 * D_G
HID = 128
OUT_DIM = 8
MAX_NUM_NODES = 1000
BATCH = (N * HID) // (HID * MAX_NUM_NODES)

NC = 2
NS = 16
CW = 100
ROWS_TOT = E // CW
ROWS_PER_TILE = ROWS_TOT // NS
NPT = N // NS

_BN = 1000


def _seg_sum_body(ha_hbm, hb_hbm, srcc_hbm, dstc_hbm, zrows_hbm, out_hbm,
                  src_v, dst_v, r0, r1, r2, r3, agg_sh, gsem, ssem):
    c = lax.axis_index("c")
    s = lax.axis_index("s")
    rows = (r0, r1, r2, r3)
    pltpu.sync_copy(srcc_hbm.at[s], src_v)
    pltpu.sync_copy(dstc_hbm.at[s], dst_v)
    pltpu.sync_copy(zrows_hbm, agg_sh.at[pl.ds(s * NPT, NPT)])
    plsc.subcore_barrier()

    def edge_loop(h_hbm):
        for b in range(2):
            pltpu.async_copy(h_hbm.at[src_v.at[b]], rows[b], gsem.at[b])

        def block(kk, carry):
            k0 = 4 * kk
            for u in range(4):
                k = k0 + u
                b = u
                b2 = (u + 2) % 4
                pltpu.make_async_copy(h_hbm.at[src_v.at[k]], rows[b],
                                      gsem.at[b]).wait()
                pltpu.async_copy(rows[b], agg_sh.at[dst_v.at[k]],
                                 ssem.at[b], add=True)

                @pl.when(k >= 2)
                def _():
                    pltpu.make_async_copy(rows[b2], agg_sh.at[dst_v.at[k]],
                                          ssem.at[b2]).wait()

                @pl.when(k + 2 < ROWS_PER_TILE)
                def _():
                    pltpu.async_copy(h_hbm.at[src_v.at[k + 2]], rows[b2],
                                     gsem.at[b2])

            return carry

        lax.fori_loop(0, ROWS_PER_TILE // 4, block, 0)
        for b in (2, 3):
            pltpu.make_async_copy(rows[b], agg_sh.at[dst_v.at[0]],
                                  ssem.at[b]).wait()

    @pl.when(c == 0)
    def _():
        edge_loop(ha_hbm)

    @pl.when(c == 1)
    def _():
        edge_loop(hb_hbm)

    plsc.subcore_barrier()
    pltpu.sync_copy(agg_sh.at[pl.ds(s * NPT, NPT)], out_hbm.at[c, s])


def _seg_sum(ha, hb, srcc, dstc, zrows):
    wh = ha.shape[1]
    mesh = plsc.VectorSubcoreMesh(core_axis_name="c", subcore_axis_name="s")
    f = pl.kernel(
        _seg_sum_body,
        out_type=jax.ShapeDtypeStruct((NC, NS, NPT, wh), jnp.float32),
        mesh=mesh,
        scratch_types=[
            pltpu.VMEM((ROWS_PER_TILE, CW), jnp.int32),
            pltpu.VMEM((ROWS_PER_TILE, CW), jnp.int32),
        ] + [pltpu.VMEM((CW, wh), jnp.float32) for _ in range(4)] + [
            pltpu.VMEM_SHARED((N, wh), jnp.float32),
            pltpu.SemaphoreType.DMA((4,)),
            pltpu.SemaphoreType.DMA((4,)),
        ],
        compiler_params=pltpu.CompilerParams(use_tc_tiling_on_sc=False),
    )
    out = f(ha, hb, srcc, dstc, zrows)
    return out[0].reshape(N, wh), out[1].reshape(N, wh)


def _deg_body(dstc_hbm, ones_hbm, zrows_hbm, out_hbm,
              dst_v, ones_v, deg_sh, sem):
    c = lax.axis_index("c")
    s = lax.axis_index("s")
    pltpu.sync_copy(dstc_hbm.at[s], dst_v)
    pltpu.sync_copy(ones_hbm, ones_v)
    pltpu.sync_copy(zrows_hbm, deg_sh.at[pl.ds(s * NPT, NPT)])
    plsc.subcore_barrier()

    half = ROWS_PER_TILE // NC

    def chunk(j, carry):
        pltpu.sync_copy(ones_v, deg_sh.at[dst_v.at[c * half + j]], add=True)
        return carry

    lax.fori_loop(0, half, chunk, 0)
    plsc.subcore_barrier()
    pltpu.sync_copy(deg_sh.at[pl.ds(s * NPT, NPT)], out_hbm.at[c, s])


def _deg_count(dstc, ones16, zrows16):
    mesh = plsc.VectorSubcoreMesh(core_axis_name="c", subcore_axis_name="s")
    f = pl.kernel(
        _deg_body,
        out_type=jax.ShapeDtypeStruct((NC, NS, NPT, 16), jnp.float32),
        mesh=mesh,
        scratch_types=[
            pltpu.VMEM((ROWS_PER_TILE, CW), jnp.int32),
            pltpu.VMEM((CW, 16), jnp.float32),
            pltpu.VMEM_SHARED((N, 16), jnp.float32),
            pltpu.SemaphoreType.DMA,
        ],
        compiler_params=pltpu.CompilerParams(use_tc_tiling_on_sc=False),
    )
    return f(dstc, ones16, zrows16).reshape(NC, N, 16)


def _layer_body(nh, na, *refs):
    h_parts = [refs[i][...] for i in range(nh)]
    agg_parts = [refs[nh + i][...] for i in range(na)]
    deg_ref, w_ref, b_ref = refs[nh + na:nh + na + 3]
    outs = refs[nh + na + 3:]
    h = jnp.concatenate(h_parts, axis=1)
    agg = jnp.concatenate(agg_parts, axis=1)
    deg = deg_ref[0, :, 0:1] + deg_ref[1, :, 0:1]
    mean = agg / jnp.maximum(deg, 1.0)
    w = w_ref[...]
    wh = h.shape[1]
    r = (jnp.dot(mean, w[wh:], preferred_element_type=jnp.float32)
         + jnp.dot(h, w[:wh], preferred_element_type=jnp.float32))
    r = jnp.maximum(r + b_ref[...], 0.0)
    if len(outs) == 1:
        outs[0][...] = r
    else:
        outs[0][...] = r[:, :HID // 2]
        outs[1][...] = r[:, HID // 2:]


def _layer(h_parts, agg_parts, deg2, wstack, bias, split_out):
    import functools
    w2 = wstack.shape[0]
    if split_out:
        out_specs = [pl.BlockSpec((_BN, HID // 2), lambda i: (i, 0)),
                     pl.BlockSpec((_BN, HID // 2), lambda i: (i, 0))]
        out_shape = [jax.ShapeDtypeStruct((N, HID // 2), jnp.float32),
                     jax.ShapeDtypeStruct((N, HID // 2), jnp.float32)]
    else:
        out_specs = pl.BlockSpec((_BN, HID), lambda i: (i, 0))
        out_shape = jax.ShapeDtypeStruct((N, HID), jnp.float32)
    in_specs = (
        [pl.BlockSpec((_BN, p.shape[1]), lambda i: (i, 0)) for p in h_parts]
        + [pl.BlockSpec((_BN, p.shape[1]), lambda i: (i, 0))
           for p in agg_parts]
        + [pl.BlockSpec((NC, _BN, 16), lambda i: (0, i, 0)),
           pl.BlockSpec((w2, HID), lambda i: (0, 0)),
           pl.BlockSpec((1, HID), lambda i: (0, 0))])
    body = functools.partial(_layer_body, len(h_parts), len(agg_parts))
    return pl.pallas_call(
        body,
        grid=(N // _BN,),
        in_specs=in_specs,
        out_specs=out_specs,
        out_shape=out_shape,
    )(*h_parts, *agg_parts, deg2, wstack, bias)


def _head1_body(hr_ref, w1_ref, b1_ref, w2_ref, b2_ref, u_out):
    t = jnp.dot(hr_ref[...], w1_ref[...],
                preferred_element_type=jnp.float32) + b1_ref[...]
    t = jnp.maximum(t, 0.0)
    u_out[...] = jnp.dot(t, w2_ref[...],
                         preferred_element_type=jnp.float32) + b2_ref[...]


def _head1(hr, m1w1, m1b1, m1w2, m1b2):
    R = BATCH * HID
    return pl.pallas_call(
        _head1_body,
        grid=(1,),
        in_specs=[
            pl.BlockSpec((R, MAX_NUM_NODES), lambda i: (0, 0)),
            pl.BlockSpec((MAX_NUM_NODES, HID), lambda i: (0, 0)),
            pl.BlockSpec((1, HID), lambda i: (0, 0)),
            pl.BlockSpec((HID, 1), lambda i: (0, 0)),
            pl.BlockSpec((1, 1), lambda i: (0, 0)),
        ],
        out_specs=pl.BlockSpec((R, 1), lambda i: (0, 0)),
        out_shape=jax.ShapeDtypeStruct((R, 1), jnp.float32),
    )(hr, m1w1, m1b1, m1w2, m1b2)


def _head2_body(u_ref, gamma_ref, beta_ref, w1_ref, b1_ref, w2_ref, b2_ref,
                out_ref):
    u = u_ref[...]
    mu = jnp.mean(u, axis=0, keepdims=True)
    var = jnp.mean((u - mu) ** 2, axis=0, keepdims=True)
    hb = (u - mu) / jnp.sqrt(var + 1e-5) * gamma_ref[...] + beta_ref[...]
    hb = jnp.maximum(hb, 0.0)
    z = jnp.dot(hb, w1_ref[...], preferred_element_type=jnp.float32)
    z = jnp.maximum(z + b1_ref[...], 0.0)
    out_ref[...] = jnp.dot(z, w2_ref[...],
                           preferred_element_type=jnp.float32) + b2_ref[...]


def _head2(u2, gamma, beta, m2w1, m2b1, m2w2, m2b2):
    return pl.pallas_call(
        _head2_body,
        grid=(1,),
        in_specs=[
            pl.BlockSpec((BATCH, HID), lambda i: (0, 0)),
            pl.BlockSpec((1, HID), lambda i: (0, 0)),
            pl.BlockSpec((1, HID), lambda i: (0, 0)),
            pl.BlockSpec((HID, HID), lambda i: (0, 0)),
            pl.BlockSpec((1, HID), lambda i: (0, 0)),
            pl.BlockSpec((HID, OUT_DIM), lambda i: (0, 0)),
            pl.BlockSpec((1, OUT_DIM), lambda i: (0, 0)),
        ],
        out_specs=pl.BlockSpec((BATCH, OUT_DIM), lambda i: (0, 0)),
        out_shape=jax.ShapeDtypeStruct((BATCH, OUT_DIM), jnp.float32),
    )(u2, gamma, beta, m2w1, m2b1, m2w2, m2b2)


def kernel(x, g0, g1, g2, edge_index, ws0, wn0, b0, ws1, wn1, b1,
           ws2, wn2, b2, ws3, wn3, b3, m1w1, m1b1, m1w2, m1b2,
           gamma, beta, m2w1, m2b1, m2w2, m2b2):
    srcc = edge_index[0].reshape(NS, ROWS_PER_TILE, CW)
    dstc = edge_index[1].reshape(NS, ROWS_PER_TILE, CW)
    zrows = jnp.zeros((NPT, HID // 2), jnp.float32)
    zrows16 = jnp.zeros((NPT, 16), jnp.float32)
    ones16 = jnp.ones((CW, 16), jnp.float32)

    deg2 = _deg_count(dstc, ones16, zrows16)

    h0 = jnp.concatenate([x, g0, g1, g2], axis=1)
    zcols = jnp.zeros((N, 2 * HID - IN_DIM), jnp.float32)
    h0p = jnp.concatenate([h0, zcols], axis=1)
    FH = HID // 2

    agg00, agg01 = _seg_sum(h0p[:, :FH], h0p[:, FH:2 * FH], srcc, dstc, zrows)
    agg02, agg03 = _seg_sum(h0p[:, 2 * FH:3 * FH], h0p[:, 3 * FH:],
                            srcc, dstc, zrows)
    wstack0 = jnp.concatenate(
        [ws0, wn0, jnp.zeros((2 * HID - IN_DIM, HID), jnp.float32)], axis=0)
    ha, hb = _layer([h0[:, :IN_DIM // 2], h0[:, IN_DIM // 2:]],
                    [agg00, agg01, agg02, agg03], deg2, wstack0,
                    b0.reshape(1, -1), split_out=True)

    h4 = None
    layers = ((ws1, wn1, b1), (ws2, wn2, b2), (ws3, wn3, b3))
    for li, (ws, wn, b) in enumerate(layers):
        agga, aggb = _seg_sum(ha, hb, srcc, dstc, zrows)
        wstack = jnp.concatenate([ws, wn], axis=0)
        last = li == len(layers) - 1
        r = _layer([ha, hb], [agga, aggb], deg2, wstack, b.reshape(1, -1),
                   split_out=not last)
        if last:
            h4 = r
        else:
            ha, hb = r

    hr = h4.reshape(BATCH * HID, MAX_NUM_NODES)
    u = _head1(hr, m1w1, m1b1.reshape(1, -1), m1w2, m1b2.reshape(1, 1))
    u2 = u.reshape(BATCH, HID)
    return _head2(u2, gamma.reshape(1, -1), beta.reshape(1, -1),
                  m2w1, m2b1.reshape(1, -1), m2w2, m2b2.reshape(1, -1))
